# baseline (speedup 1.0000x reference)
import functools

import jax
import jax.numpy as jnp
from jax.experimental import pallas as pl
from jax.experimental.pallas import tpu as pltpu


def _mm(a, b):
    return jax.lax.dot_general(a, b, (((1,), (0,)), ((), ())),
                               preferred_element_type=jnp.float32)


def _nt(a, b, precision=None):
    return jax.lax.dot_general(a, b, (((1,), (1,)), ((), ())),
                               preferred_element_type=jnp.float32,
                               precision=precision)


def _dis(deg):
    safe = jnp.where(deg > 0, deg, 1.0)
    return jnp.where(deg > 0, jax.lax.rsqrt(safe), 0.0)


def _d2_tile(zn2_i, z_j, sqc_i, sqr_j):
    return jnp.maximum(_nt(zn2_i, z_j) + sqc_i + sqr_j, 0.0)


def _row_norms(z):
    zz = z * z
    ones = jnp.ones((1, z.shape[1]), jnp.float32)
    return _nt(ones, zz, precision=jax.lax.Precision.HIGHEST)


def _bn_lrelu(pre, g, bb):
    m = jnp.mean(pre, axis=0, keepdims=True)
    v = jnp.mean((pre - m) ** 2, axis=0, keepdims=True)
    hn = (pre - m) / jnp.sqrt(v + 1e-5) * g + bb
    return jnp.where(hn >= 0, hn, 0.01 * hn)


def _ti(n):
    return 512 if n % 512 == 0 and n >= 1024 else n // 2


def _ts(n):
    return 1024 if n % 1024 == 0 and n >= 2048 else _ti(n)


def _tj(n):
    return 2048 if n % 2048 == 0 and n >= 4096 else _ti(n)


def _prep_kernel(ni_ref, g_ref, b_ref, mw_ref, mb_ref, w2n_ref, w_ref,
                 sqc_ref, sqr_ref):
    z = ni_ref[...]
    m = jnp.mean(z, axis=0, keepdims=True)
    v = jnp.mean((z - m) ** 2, axis=0, keepdims=True)
    zn = (z - m) / jnp.sqrt(v + 1e-5) * g_ref[...] + b_ref[...]
    w = _mm(zn, mw_ref[...]) + mb_ref[...]
    w_ref[...] = w
    w2n_ref[...] = w * -2.0
    sqc_ref[...] = jnp.sum(w * w, axis=1, keepdims=True)
    sqr_ref[...] = _row_norms(w)


def _prep(noimg, g, b, mw, mb):
    n = noimg.shape[0]
    kw = mw.shape[1]
    return pl.pallas_call(
        _prep_kernel,
        out_shape=(jax.ShapeDtypeStruct((n, kw), jnp.float32),
                   jax.ShapeDtypeStruct((n, kw), jnp.float32),
                   jax.ShapeDtypeStruct((n, 1), jnp.float32),
                   jax.ShapeDtypeStruct((1, n), jnp.float32)),
    )(noimg, g, b, mw, mb)


def _mat_a_kernel(s2_ref, w2n_ref, w_ref, sqc_ref, sqr_ref, x_ref, a_ref,
                  deg_ref, v0_ref):
    j = pl.program_id(1)
    nj = pl.num_programs(1)
    d2 = _d2_tile(w2n_ref[...], w_ref[...], sqc_ref[...], sqr_ref[...])
    wm = (jnp.exp(d2 * (-1.0 / 16.0)) + 1.0) * 0.5
    a = wm * s2_ref[...]
    a_ref[...] = a.astype(jnp.bfloat16)
    rs = jnp.sum(a, axis=1, keepdims=True)

    @pl.when(j == 0)
    def _():
        deg_ref[...] = rs

    @pl.when(j > 0)
    def _():
        deg_ref[...] += rs

    @pl.when(j == nj - 1)
    def _():
        v0_ref[...] = (_dis(deg_ref[...]) * x_ref[...]).astype(jnp.bfloat16)


def _mat_a0_kernel(w2n_ref, w_ref, sqwc_ref, sqwr_ref, h2n_ref, h_ref,
                   sqhc_ref, sqhr_ref, x_ref, a_ref, deg_ref, v0_ref):
    j = pl.program_id(1)
    nj = pl.num_programs(1)
    d2w = _d2_tile(w2n_ref[...], w_ref[...], sqwc_ref[...], sqwr_ref[...])
    wm = (jnp.exp(d2w * (-1.0 / 16.0)) + 1.0) * 0.5
    d2h = _d2_tile(h2n_ref[...], h_ref[...], sqhc_ref[...], sqhr_ref[...])
    a = wm * jnp.exp(d2h * (-1.0 / 256.0))
    a_ref[...] = a.astype(jnp.bfloat16)
    rs = jnp.sum(a, axis=1, keepdims=True)

    @pl.when(j == 0)
    def _():
        deg_ref[...] = rs

    @pl.when(j > 0)
    def _():
        deg_ref[...] += rs

    @pl.when(j == nj - 1)
    def _():
        v0_ref[...] = (_dis(deg_ref[...]) * x_ref[...]).astype(jnp.bfloat16)


def _mat_a(S2, w2n, w, sqwc, sqwr, x):
    n = S2.shape[0]
    ti, tj = _ti(n), _tj(n)
    kw = w.shape[1]
    d = x.shape[1]
    return pl.pallas_call(
        _mat_a_kernel,
        grid=(n // ti, n // tj),
        in_specs=[
            pl.BlockSpec((ti, tj), lambda i, j: (i, j)),
            pl.BlockSpec((ti, kw), lambda i, j: (i, 0)),
            pl.BlockSpec((tj, kw), lambda i, j: (j, 0)),
            pl.BlockSpec((ti, 1), lambda i, j: (i, 0)),
            pl.BlockSpec((1, tj), lambda i, j: (0, j)),
            pl.BlockSpec((ti, d), lambda i, j: (i, 0)),
        ],
        out_specs=(pl.BlockSpec((ti, tj), lambda i, j: (i, j)),
                   pl.BlockSpec((ti, 1), lambda i, j: (i, 0)),
                   pl.BlockSpec((ti, d), lambda i, j: (i, 0))),
        out_shape=(jax.ShapeDtypeStruct((n, n), jnp.bfloat16),
                   jax.ShapeDtypeStruct((n, 1), jnp.float32),
                   jax.ShapeDtypeStruct((n, d), jnp.bfloat16)),
    )(S2, w2n, w, sqwc, sqwr, x)


def _mat_a0(w2n, w, sqwc, sqwr, h2n, h, sqhc, sqhr, x):
    n = w.shape[0]
    ti, tj = _ti(n), _tj(n)
    kw = w.shape[1]
    kh = h.shape[1]
    d = x.shape[1]
    return pl.pallas_call(
        _mat_a0_kernel,
        grid=(n // ti, n // tj),
        in_specs=[
            pl.BlockSpec((ti, kw), lambda i, j: (i, 0)),
            pl.BlockSpec((tj, kw), lambda i, j: (j, 0)),
            pl.BlockSpec((ti, 1), lambda i, j: (i, 0)),
            pl.BlockSpec((1, tj), lambda i, j: (0, j)),
            pl.BlockSpec((ti, kh), lambda i, j: (i, 0)),
            pl.BlockSpec((tj, kh), lambda i, j: (j, 0)),
            pl.BlockSpec((ti, 1), lambda i, j: (i, 0)),
            pl.BlockSpec((1, tj), lambda i, j: (0, j)),
            pl.BlockSpec((ti, d), lambda i, j: (i, 0)),
        ],
        out_specs=(pl.BlockSpec((ti, tj), lambda i, j: (i, j)),
                   pl.BlockSpec((ti, 1), lambda i, j: (i, 0)),
                   pl.BlockSpec((ti, d), lambda i, j: (i, 0))),
        out_shape=(jax.ShapeDtypeStruct((n, n), jnp.bfloat16),
                   jax.ShapeDtypeStruct((n, 1), jnp.float32),
                   jax.ShapeDtypeStruct((n, d), jnp.bfloat16)),
    )(w2n, w, sqwc, sqwr, h2n, h, sqhc, sqhr, x)


def _layer_t1(i, ti, a_ref, deg_ref, v_ref, t1_scr, vt1_scr):
    acc = _mm(a_ref[...], v_ref[...])
    di = _dis(deg_ref[...])
    y = acc * (-di)
    t1_scr[pl.ds(i * ti, ti), :] = y
    vt1_scr[pl.ds(i * ti, ti), :] = (di * y).astype(jnp.bfloat16)


def _layer_z(k, ti, a_ref, deg_ref, x_ref, w0_ref, w1_ref, w2_ref, b_ref,
             t1_scr, vt1_scr, pre_scr):
    acc = _mm(a_ref[...], vt1_scr[...])
    z = acc * (-_dis(deg_ref[...]))
    x0 = x_ref[...]
    tx2 = 2.0 * z - x0
    pre = (_mm(x0, w0_ref[...]) + _mm(t1_scr[pl.ds(k * ti, ti), :],
                                      w1_ref[...])
           + _mm(tx2, w2_ref[...]) + b_ref[...])
    pre_scr[pl.ds(k * ti, ti), :] = pre


def _layer_phases(ti, a_ref, deg_ref, v_ref, x_ref, w0_ref, w1_ref, w2_ref,
                  b_ref, t1_scr, vt1_scr, pre_scr):
    i = pl.program_id(0)
    ns = pl.num_programs(0) // 2

    @pl.when(i < ns)
    def _():
        _layer_t1(i, ti, a_ref, deg_ref, v_ref, t1_scr, vt1_scr)

    @pl.when(i >= ns)
    def _():
        _layer_z(i - ns, ti, a_ref, deg_ref, x_ref, w0_ref, w1_ref, w2_ref,
                 b_ref, t1_scr, vt1_scr, pre_scr)

    return i == 2 * ns - 1


def _layer_v_kernel(ti, a_ref, deg_ref, v_ref, x_ref, w0_ref, w1_ref, w2_ref,
                    b_ref, g_ref, bb_ref, degf_ref, h_ref, vh_ref, t1_scr,
                    vt1_scr, pre_scr):
    done = _layer_phases(ti, a_ref, deg_ref, v_ref, x_ref, w0_ref, w1_ref,
                         w2_ref, b_ref, t1_scr, vt1_scr, pre_scr)

    @pl.when(done)
    def _():
        h = _bn_lrelu(pre_scr[...], g_ref[...], bb_ref[...])
        h_ref[...] = h
        vh_ref[...] = (_dis(degf_ref[...]) * h).astype(jnp.bfloat16)


def _layer_aug_kernel(ti, a_ref, deg_ref, v_ref, x_ref, w0_ref, w1_ref,
                      w2_ref, b_ref, g_ref, bb_ref, h_ref, h2n_ref, sqc_ref,
                      sqr_ref, t1_scr, vt1_scr, pre_scr):
    done = _layer_phases(ti, a_ref, deg_ref, v_ref, x_ref, w0_ref, w1_ref,
                         w2_ref, b_ref, t1_scr, vt1_scr, pre_scr)

    @pl.when(done)
    def _():
        h = _bn_lrelu(pre_scr[...], g_ref[...], bb_ref[...])
        h_ref[...] = h
        h2n_ref[...] = h * -2.0
        sqc_ref[...] = jnp.sum(h * h, axis=1, keepdims=True)
        sqr_ref[...] = _row_norms(h)


def _layer_head_kernel(ti, a_ref, deg_ref, v_ref, x_ref, w0_ref, w1_ref,
                       w2_ref, b_ref, g_ref, bb_ref, p1w_ref, p1b_ref,
                       gp_ref, bp_ref, p2w_ref, p2b_ref, out_ref, t1_scr,
                       vt1_scr, pre_scr):
    done = _layer_phases(ti, a_ref, deg_ref, v_ref, x_ref, w0_ref, w1_ref,
                         w2_ref, b_ref, t1_scr, vt1_scr, pre_scr)

    @pl.when(done)
    def _():
        h = _bn_lrelu(pre_scr[...], g_ref[...], bb_ref[...])
        p = jnp.maximum(_mm(h, p1w_ref[...]) + p1b_ref[...], 0.0)
        m = jnp.mean(p, axis=0, keepdims=True)
        v = jnp.mean((p - m) ** 2, axis=0, keepdims=True)
        p = (p - m) / jnp.sqrt(v + 1e-5) * gp_ref[...] + bp_ref[...]
        out_ref[...] = jnp.maximum(_mm(p, p2w_ref[...]) + p2b_ref[...], 0.0)


def _full(shape):
    return pl.BlockSpec(shape, lambda i: tuple(0 for _ in shape))


def _layer_specs(n, ti, d):
    ns = n // ti

    def smap(i):
        return (jnp.where(i < ns, i, i - ns), 0)

    def xmap(i):
        return (jnp.where(i < ns, 0, i - ns), 0)

    return [
        pl.BlockSpec((ti, n), smap),
        pl.BlockSpec((ti, 1), smap),
        _full((n, d)),
        pl.BlockSpec((ti, d), xmap),
    ]


def _layer_scratch(n, d):
    return [pltpu.VMEM((n, d), jnp.float32),
            pltpu.VMEM((n, d), jnp.bfloat16),
            pltpu.VMEM((n, d), jnp.float32)]


def _layer_v(a, deg, v, x0, w0, w1, w2, b, g, bb, degf):
    n, d = x0.shape
    ti = _ts(n)
    return pl.pallas_call(
        functools.partial(_layer_v_kernel, ti),
        grid=(2 * (n // ti),),
        in_specs=_layer_specs(n, ti, d) + [
            _full(w0.shape), _full(w1.shape), _full(w2.shape),
            _full(b.shape), _full(g.shape), _full(bb.shape),
            _full(degf.shape),
        ],
        out_specs=(_full((n, d)), _full((n, d))),
        out_shape=(jax.ShapeDtypeStruct((n, d), jnp.float32),
                   jax.ShapeDtypeStruct((n, d), jnp.bfloat16)),
        scratch_shapes=_layer_scratch(n, d),
    )(a, deg, v, x0, w0, w1, w2, b, g, bb, degf)


def _layer_aug(a, deg, v, x0, w0, w1, w2, b, g, bb):
    n, d = x0.shape
    ti = _ts(n)
    return pl.pallas_call(
        functools.partial(_layer_aug_kernel, ti),
        grid=(2 * (n // ti),),
        in_specs=_layer_specs(n, ti, d) + [
            _full(w0.shape), _full(w1.shape), _full(w2.shape),
            _full(b.shape), _full(g.shape), _full(bb.shape),
        ],
        out_specs=(_full((n, d)), _full((n, d)), _full((n, 1)),
                   _full((1, n))),
        out_shape=(jax.ShapeDtypeStruct((n, d), jnp.float32),
                   jax.ShapeDtypeStruct((n, d), jnp.float32),
                   jax.ShapeDtypeStruct((n, 1), jnp.float32),
                   jax.ShapeDtypeStruct((1, n), jnp.float32)),
        scratch_shapes=_layer_scratch(n, d),
    )(a, deg, v, x0, w0, w1, w2, b, g, bb)


def _layer_head(a, deg, v, x0, w0, w1, w2, b, g, bb, p1w, p1b, gp, bp, p2w,
                p2b):
    n, d = x0.shape
    ti = _ts(n)
    nc = p2w.shape[1]
    return pl.pallas_call(
        functools.partial(_layer_head_kernel, ti),
        grid=(2 * (n // ti),),
        in_specs=_layer_specs(n, ti, d) + [
            _full(w0.shape), _full(w1.shape), _full(w2.shape),
            _full(b.shape), _full(g.shape), _full(bb.shape),
            _full(p1w.shape), _full(p1b.shape), _full(gp.shape),
            _full(bp.shape), _full(p2w.shape), _full(p2b.shape),
        ],
        out_specs=_full((n, nc)),
        out_shape=jax.ShapeDtypeStruct((n, nc), jnp.float32),
        scratch_shapes=_layer_scratch(n, d),
    )(a, deg, v, x0, w0, w1, w2, b, g, bb, p1w, p1b, gp, bp, p2w, p2b)


def kernel(x, S2, no_image_feature, bn3_g, bn3_b, mlp_w, mlp_b, c1_w0, c1_w1,
           c1_w2, c1_b, c2_w0, c2_w1, c2_w2, c2_b, bn1_g, bn1_b, bn2_g, bn2_b,
           p1_w, p1_b, bnp_g, bnp_b, p2_w, p2_b):
    r2 = lambda a: a.reshape(1, -1)

    w2n, w, sqwc, sqwr = _prep(no_image_feature, r2(bn3_g), r2(bn3_b), mlp_w,
                               r2(mlp_b))

    a, deg, v0 = _mat_a(S2, w2n, w, sqwc, sqwr, x)
    h, vh = _layer_v(a, deg, v0, x, c1_w0, c1_w1, c1_w2, r2(c1_b), r2(bn1_g),
                     r2(bn1_b), deg)
    h2, h2n, sqhc, sqhr = _layer_aug(a, deg, vh, h, c2_w0, c2_w1, c2_w2,
                                     r2(c2_b), r2(bn2_g), r2(bn2_b))

    a0, deg0, u0 = _mat_a0(w2n, w, sqwc, sqwr, h2n, h2, sqhc, sqhr, x)
    g1, vg1 = _layer_v(a0, deg0, u0, x, c1_w0, c1_w1, c1_w2, r2(c1_b),
                       r2(bn1_g), r2(bn1_b), deg0)
    return _layer_head(a0, deg0, vg1, g1, c2_w0, c2_w1, c2_w2, r2(c2_b),
                       r2(bn2_g), r2(bn2_b), p1_w, r2(p1_b), r2(bnp_g),
                       r2(bnp_b), p2_w, r2(p2_b))


# speedup vs baseline: 1.0155x; 1.0155x over previous
import functools

import jax
import jax.numpy as jnp
from jax.experimental import pallas as pl
from jax.experimental.pallas import tpu as pltpu


def _mm(a, b):
    return jax.lax.dot_general(a, b, (((1,), (0,)), ((), ())),
                               preferred_element_type=jnp.float32)


def _nt(a, b, precision=None):
    return jax.lax.dot_general(a, b, (((1,), (1,)), ((), ())),
                               preferred_element_type=jnp.float32,
                               precision=precision)


def _dis(deg):
    safe = jnp.where(deg > 0, deg, 1.0)
    return jnp.where(deg > 0, jax.lax.rsqrt(safe), 0.0)


def _d2_tile(zn2_i, z_j, sqc_i, sqr_j):
    return jnp.maximum(_nt(zn2_i, z_j) + sqc_i + sqr_j, 0.0)


def _row_norms(z):
    zz = z * z
    ones = jnp.ones((1, z.shape[1]), jnp.float32)
    return _nt(ones, zz, precision=jax.lax.Precision.HIGHEST)


def _bn_lrelu(pre, g, bb):
    m = jnp.mean(pre, axis=0, keepdims=True)
    v = jnp.mean((pre - m) ** 2, axis=0, keepdims=True)
    hn = (pre - m) / jnp.sqrt(v + 1e-5) * g + bb
    return jnp.where(hn >= 0, hn, 0.01 * hn)


def _ti(n):
    return 512 if n % 512 == 0 and n >= 1024 else n // 2


def _ts(n):
    return 2048 if n % 2048 == 0 and n >= 4096 else _ti(n)


def _tj(n):
    return n


def _prep_kernel(ni_ref, g_ref, b_ref, mw_ref, mb_ref, w2n_ref, w_ref,
                 sqc_ref, sqr_ref):
    z = ni_ref[...]
    m = jnp.mean(z, axis=0, keepdims=True)
    v = jnp.mean((z - m) ** 2, axis=0, keepdims=True)
    zn = (z - m) / jnp.sqrt(v + 1e-5) * g_ref[...] + b_ref[...]
    w = _mm(zn, mw_ref[...]) + mb_ref[...]
    w_ref[...] = w
    w2n_ref[...] = w * -2.0
    sqc_ref[...] = jnp.sum(w * w, axis=1, keepdims=True)
    sqr_ref[...] = _row_norms(w)


def _prep(noimg, g, b, mw, mb):
    n = noimg.shape[0]
    kw = mw.shape[1]
    return pl.pallas_call(
        _prep_kernel,
        out_shape=(jax.ShapeDtypeStruct((n, kw), jnp.float32),
                   jax.ShapeDtypeStruct((n, kw), jnp.float32),
                   jax.ShapeDtypeStruct((n, 1), jnp.float32),
                   jax.ShapeDtypeStruct((1, n), jnp.float32)),
    )(noimg, g, b, mw, mb)


def _mat_a_kernel(s2_ref, w2n_ref, w_ref, sqc_ref, sqr_ref, x_ref, a_ref,
                  deg_ref, v0_ref):
    j = pl.program_id(1)
    nj = pl.num_programs(1)
    d2 = _d2_tile(w2n_ref[...], w_ref[...], sqc_ref[...], sqr_ref[...])
    wm = (jnp.exp(d2 * (-1.0 / 16.0)) + 1.0) * 0.5
    a = wm * s2_ref[...]
    a_ref[...] = a.astype(jnp.bfloat16)
    rs = jnp.sum(a, axis=1, keepdims=True)

    @pl.when(j == 0)
    def _():
        deg_ref[...] = rs

    @pl.when(j > 0)
    def _():
        deg_ref[...] += rs

    @pl.when(j == nj - 1)
    def _():
        v0_ref[...] = (_dis(deg_ref[...]) * x_ref[...]).astype(jnp.bfloat16)


def _mat_a0_kernel(w2n_ref, w_ref, sqwc_ref, sqwr_ref, h2n_ref, h_ref,
                   sqhc_ref, sqhr_ref, x_ref, a_ref, deg_ref, v0_ref):
    j = pl.program_id(1)
    nj = pl.num_programs(1)
    d2w = _d2_tile(w2n_ref[...], w_ref[...], sqwc_ref[...], sqwr_ref[...])
    wm = (jnp.exp(d2w * (-1.0 / 16.0)) + 1.0) * 0.5
    d2h = _d2_tile(h2n_ref[...], h_ref[...], sqhc_ref[...], sqhr_ref[...])
    a = wm * jnp.exp(d2h * (-1.0 / 256.0))
    a_ref[...] = a.astype(jnp.bfloat16)
    rs = jnp.sum(a, axis=1, keepdims=True)

    @pl.when(j == 0)
    def _():
        deg_ref[...] = rs

    @pl.when(j > 0)
    def _():
        deg_ref[...] += rs

    @pl.when(j == nj - 1)
    def _():
        v0_ref[...] = (_dis(deg_ref[...]) * x_ref[...]).astype(jnp.bfloat16)


def _mat_a(S2, w2n, w, sqwc, sqwr, x):
    n = S2.shape[0]
    ti, tj = _ti(n), _tj(n)
    kw = w.shape[1]
    d = x.shape[1]
    return pl.pallas_call(
        _mat_a_kernel,
        grid=(n // ti, n // tj),
        in_specs=[
            pl.BlockSpec((ti, tj), lambda i, j: (i, j)),
            pl.BlockSpec((ti, kw), lambda i, j: (i, 0)),
            pl.BlockSpec((tj, kw), lambda i, j: (j, 0)),
            pl.BlockSpec((ti, 1), lambda i, j: (i, 0)),
            pl.BlockSpec((1, tj), lambda i, j: (0, j)),
            pl.BlockSpec((ti, d), lambda i, j: (i, 0)),
        ],
        out_specs=(pl.BlockSpec((ti, tj), lambda i, j: (i, j)),
                   pl.BlockSpec((ti, 1), lambda i, j: (i, 0)),
                   pl.BlockSpec((ti, d), lambda i, j: (i, 0))),
        out_shape=(jax.ShapeDtypeStruct((n, n), jnp.bfloat16),
                   jax.ShapeDtypeStruct((n, 1), jnp.float32),
                   jax.ShapeDtypeStruct((n, d), jnp.bfloat16)),
    )(S2, w2n, w, sqwc, sqwr, x)


def _mat_a0(w2n, w, sqwc, sqwr, h2n, h, sqhc, sqhr, x):
    n = w.shape[0]
    ti, tj = _ti(n), _tj(n)
    kw = w.shape[1]
    kh = h.shape[1]
    d = x.shape[1]
    return pl.pallas_call(
        _mat_a0_kernel,
        grid=(n // ti, n // tj),
        in_specs=[
            pl.BlockSpec((ti, kw), lambda i, j: (i, 0)),
            pl.BlockSpec((tj, kw), lambda i, j: (j, 0)),
            pl.BlockSpec((ti, 1), lambda i, j: (i, 0)),
            pl.BlockSpec((1, tj), lambda i, j: (0, j)),
            pl.BlockSpec((ti, kh), lambda i, j: (i, 0)),
            pl.BlockSpec((tj, kh), lambda i, j: (j, 0)),
            pl.BlockSpec((ti, 1), lambda i, j: (i, 0)),
            pl.BlockSpec((1, tj), lambda i, j: (0, j)),
            pl.BlockSpec((ti, d), lambda i, j: (i, 0)),
        ],
        out_specs=(pl.BlockSpec((ti, tj), lambda i, j: (i, j)),
                   pl.BlockSpec((ti, 1), lambda i, j: (i, 0)),
                   pl.BlockSpec((ti, d), lambda i, j: (i, 0))),
        out_shape=(jax.ShapeDtypeStruct((n, n), jnp.bfloat16),
                   jax.ShapeDtypeStruct((n, 1), jnp.float32),
                   jax.ShapeDtypeStruct((n, d), jnp.bfloat16)),
    )(w2n, w, sqwc, sqwr, h2n, h, sqhc, sqhr, x)


def _layer_t1(i, ti, a_ref, deg_ref, v_ref, t1_scr, vt1_scr):
    acc = _mm(a_ref[...], v_ref[...])
    di = _dis(deg_ref[...])
    y = acc * (-di)
    t1_scr[pl.ds(i * ti, ti), :] = y
    vt1_scr[pl.ds(i * ti, ti), :] = (di * y).astype(jnp.bfloat16)


def _layer_z(k, ti, a_ref, deg_ref, x_ref, w0_ref, w1_ref, w2_ref, b_ref,
             t1_scr, vt1_scr, pre_scr):
    acc = _mm(a_ref[...], vt1_scr[...])
    z = acc * (-_dis(deg_ref[...]))
    x0 = x_ref[...]
    tx2 = 2.0 * z - x0
    pre = (_mm(x0, w0_ref[...]) + _mm(t1_scr[pl.ds(k * ti, ti), :],
                                      w1_ref[...])
           + _mm(tx2, w2_ref[...]) + b_ref[...])
    pre_scr[pl.ds(k * ti, ti), :] = pre


def _layer_phases(ti, a_ref, deg_ref, v_ref, x_ref, w0_ref, w1_ref, w2_ref,
                  b_ref, t1_scr, vt1_scr, pre_scr):
    i = pl.program_id(0)
    ns = pl.num_programs(0) // 2

    @pl.when(i < ns)
    def _():
        _layer_t1(i, ti, a_ref, deg_ref, v_ref, t1_scr, vt1_scr)

    @pl.when(i >= ns)
    def _():
        _layer_z(i - ns, ti, a_ref, deg_ref, x_ref, w0_ref, w1_ref, w2_ref,
                 b_ref, t1_scr, vt1_scr, pre_scr)

    return i == 2 * ns - 1


def _layer_v_kernel(ti, a_ref, deg_ref, v_ref, x_ref, w0_ref, w1_ref, w2_ref,
                    b_ref, g_ref, bb_ref, degf_ref, h_ref, vh_ref, t1_scr,
                    vt1_scr, pre_scr):
    done = _layer_phases(ti, a_ref, deg_ref, v_ref, x_ref, w0_ref, w1_ref,
                         w2_ref, b_ref, t1_scr, vt1_scr, pre_scr)

    @pl.when(done)
    def _():
        h = _bn_lrelu(pre_scr[...], g_ref[...], bb_ref[...])
        h_ref[...] = h
        vh_ref[...] = (_dis(degf_ref[...]) * h).astype(jnp.bfloat16)


def _layer_aug_kernel(ti, a_ref, deg_ref, v_ref, x_ref, w0_ref, w1_ref,
                      w2_ref, b_ref, g_ref, bb_ref, h_ref, h2n_ref, sqc_ref,
                      sqr_ref, t1_scr, vt1_scr, pre_scr):
    done = _layer_phases(ti, a_ref, deg_ref, v_ref, x_ref, w0_ref, w1_ref,
                         w2_ref, b_ref, t1_scr, vt1_scr, pre_scr)

    @pl.when(done)
    def _():
        h = _bn_lrelu(pre_scr[...], g_ref[...], bb_ref[...])
        h_ref[...] = h
        h2n_ref[...] = h * -2.0
        sqc_ref[...] = jnp.sum(h * h, axis=1, keepdims=True)
        sqr_ref[...] = _row_norms(h)


def _layer_head_kernel(ti, a_ref, deg_ref, v_ref, x_ref, w0_ref, w1_ref,
                       w2_ref, b_ref, g_ref, bb_ref, p1w_ref, p1b_ref,
                       gp_ref, bp_ref, p2w_ref, p2b_ref, out_ref, t1_scr,
                       vt1_scr, pre_scr):
    done = _layer_phases(ti, a_ref, deg_ref, v_ref, x_ref, w0_ref, w1_ref,
                         w2_ref, b_ref, t1_scr, vt1_scr, pre_scr)

    @pl.when(done)
    def _():
        h = _bn_lrelu(pre_scr[...], g_ref[...], bb_ref[...])
        p = jnp.maximum(_mm(h, p1w_ref[...]) + p1b_ref[...], 0.0)
        m = jnp.mean(p, axis=0, keepdims=True)
        v = jnp.mean((p - m) ** 2, axis=0, keepdims=True)
        p = (p - m) / jnp.sqrt(v + 1e-5) * gp_ref[...] + bp_ref[...]
        out_ref[...] = jnp.maximum(_mm(p, p2w_ref[...]) + p2b_ref[...], 0.0)


def _full(shape):
    return pl.BlockSpec(shape, lambda i: tuple(0 for _ in shape))


def _layer_specs(n, ti, d):
    ns = n // ti

    def smap(i):
        return (jnp.where(i < ns, i, i - ns), 0)

    def xmap(i):
        return (jnp.where(i < ns, 0, i - ns), 0)

    return [
        pl.BlockSpec((ti, n), smap),
        pl.BlockSpec((ti, 1), smap),
        _full((n, d)),
        pl.BlockSpec((ti, d), xmap),
    ]


def _layer_scratch(n, d):
    return [pltpu.VMEM((n, d), jnp.float32),
            pltpu.VMEM((n, d), jnp.bfloat16),
            pltpu.VMEM((n, d), jnp.float32)]


def _layer_v(a, deg, v, x0, w0, w1, w2, b, g, bb, degf):
    n, d = x0.shape
    ti = _ts(n)
    return pl.pallas_call(
        functools.partial(_layer_v_kernel, ti),
        grid=(2 * (n // ti),),
        in_specs=_layer_specs(n, ti, d) + [
            _full(w0.shape), _full(w1.shape), _full(w2.shape),
            _full(b.shape), _full(g.shape), _full(bb.shape),
            _full(degf.shape),
        ],
        out_specs=(_full((n, d)), _full((n, d))),
        out_shape=(jax.ShapeDtypeStruct((n, d), jnp.float32),
                   jax.ShapeDtypeStruct((n, d), jnp.bfloat16)),
        scratch_shapes=_layer_scratch(n, d),
    )(a, deg, v, x0, w0, w1, w2, b, g, bb, degf)


def _layer_aug(a, deg, v, x0, w0, w1, w2, b, g, bb):
    n, d = x0.shape
    ti = _ts(n)
    return pl.pallas_call(
        functools.partial(_layer_aug_kernel, ti),
        grid=(2 * (n // ti),),
        in_specs=_layer_specs(n, ti, d) + [
            _full(w0.shape), _full(w1.shape), _full(w2.shape),
            _full(b.shape), _full(g.shape), _full(bb.shape),
        ],
        out_specs=(_full((n, d)), _full((n, d)), _full((n, 1)),
                   _full((1, n))),
        out_shape=(jax.ShapeDtypeStruct((n, d), jnp.float32),
                   jax.ShapeDtypeStruct((n, d), jnp.float32),
                   jax.ShapeDtypeStruct((n, 1), jnp.float32),
                   jax.ShapeDtypeStruct((1, n), jnp.float32)),
        scratch_shapes=_layer_scratch(n, d),
    )(a, deg, v, x0, w0, w1, w2, b, g, bb)


def _layer_head(a, deg, v, x0, w0, w1, w2, b, g, bb, p1w, p1b, gp, bp, p2w,
                p2b):
    n, d = x0.shape
    ti = _ts(n)
    nc = p2w.shape[1]
    return pl.pallas_call(
        functools.partial(_layer_head_kernel, ti),
        grid=(2 * (n // ti),),
        in_specs=_layer_specs(n, ti, d) + [
            _full(w0.shape), _full(w1.shape), _full(w2.shape),
            _full(b.shape), _full(g.shape), _full(bb.shape),
            _full(p1w.shape), _full(p1b.shape), _full(gp.shape),
            _full(bp.shape), _full(p2w.shape), _full(p2b.shape),
        ],
        out_specs=_full((n, nc)),
        out_shape=jax.ShapeDtypeStruct((n, nc), jnp.float32),
        scratch_shapes=_layer_scratch(n, d),
    )(a, deg, v, x0, w0, w1, w2, b, g, bb, p1w, p1b, gp, bp, p2w, p2b)


def kernel(x, S2, no_image_feature, bn3_g, bn3_b, mlp_w, mlp_b, c1_w0, c1_w1,
           c1_w2, c1_b, c2_w0, c2_w1, c2_w2, c2_b, bn1_g, bn1_b, bn2_g, bn2_b,
           p1_w, p1_b, bnp_g, bnp_b, p2_w, p2_b):
    r2 = lambda a: a.reshape(1, -1)

    w2n, w, sqwc, sqwr = _prep(no_image_feature, r2(bn3_g), r2(bn3_b), mlp_w,
                               r2(mlp_b))

    a, deg, v0 = _mat_a(S2, w2n, w, sqwc, sqwr, x)
    h, vh = _layer_v(a, deg, v0, x, c1_w0, c1_w1, c1_w2, r2(c1_b), r2(bn1_g),
                     r2(bn1_b), deg)
    h2, h2n, sqhc, sqhr = _layer_aug(a, deg, vh, h, c2_w0, c2_w1, c2_w2,
                                     r2(c2_b), r2(bn2_g), r2(bn2_b))

    a0, deg0, u0 = _mat_a0(w2n, w, sqwc, sqwr, h2n, h2, sqhc, sqhr, x)
    g1, vg1 = _layer_v(a0, deg0, u0, x, c1_w0, c1_w1, c1_w2, r2(c1_b),
                       r2(bn1_g), r2(bn1_b), deg0)
    return _layer_head(a0, deg0, vg1, g1, c2_w0, c2_w1, c2_w2, r2(c2_b),
                       r2(bn2_g), r2(bn2_b), p1_w, r2(p1_b), r2(bnp_g),
                       r2(bnp_b), p2_w, r2(p2_b))
